# baseline (speedup 1.0000x reference)
import functools

import jax
import jax.numpy as jnp
from jax import lax
from jax.experimental import pallas as pl
from jax.experimental.pallas import tpu as pltpu
from jax.experimental.pallas import tpu_sc as plsc

_BATCH = 16384
_DIM = 64
_LANES = 16
_IDX_CHUNK = 128


@functools.cache
def _build():
    info = plsc.get_sparse_core_info()
    nc, ns = info.num_cores, info.num_subcores
    nw = nc * ns
    bpw = _BATCH // nw
    nchunks = bpw // _IDX_CHUNK
    mesh = plsc.VectorSubcoreMesh(core_axis_name="c", subcore_axis_name="s")

    @functools.partial(
        pl.kernel,
        mesh=mesh,
        out_type=jax.ShapeDtypeStruct((nw, _LANES), jnp.float32),
        compiler_params=pltpu.CompilerParams(use_tc_tiling_on_sc=False),
        scratch_types=[
            pltpu.VMEM((nchunks, _IDX_CHUNK), jnp.int32),
            pltpu.VMEM((bpw, _DIM), jnp.float32),
            pltpu.VMEM((bpw, _DIM), jnp.float32),
            pltpu.VMEM((_LANES,), jnp.float32),
            pltpu.SemaphoreType.DMA,
        ],
    )
    def sc_kernel(emb_hbm, lab_hbm, cent_hbm, out_hbm,
                  idx_v, cent_v, emb_v, acc_v, sem):
        wid = lax.axis_index("s") * nc + lax.axis_index("c")

        pltpu.sync_copy(lab_hbm.at[wid], idx_v)

        copies = []
        for t in range(nchunks):
            copies.append(pltpu.async_copy(
                cent_hbm.at[idx_v.at[t]],
                cent_v.at[pl.ds(t * _IDX_CHUNK, _IDX_CHUNK)],
                sem))
        pltpu.sync_copy(emb_hbm.at[wid], emb_v)
        for c in copies:
            c.wait()

        zero = jnp.zeros((_LANES,), jnp.float32)

        def body(i, accs):
            out = []
            for j in range(_DIM // _LANES):
                e = emb_v[i, pl.ds(j * _LANES, _LANES)]
                c = cent_v[i, pl.ds(j * _LANES, _LANES)]
                d = e - c
                out.append(accs[j] + d * d)
            return tuple(out)

        accs = lax.fori_loop(0, bpw, body, (zero,) * (_DIM // _LANES))
        acc_v[...] = (accs[0] + accs[1]) + (accs[2] + accs[3])
        pltpu.sync_copy(acc_v, out_hbm.at[wid])

    return sc_kernel, nw, bpw, nchunks


def kernel(embeddings, labels, centers):
    sc_kernel, nw, bpw, nchunks = _build()
    lab = labels.astype(jnp.int32).reshape(nw, nchunks, _IDX_CHUNK)
    emb = embeddings.reshape(nw, bpw, _DIM)
    partials = sc_kernel(emb, lab, centers)
    return jnp.sum(partials) / _BATCH
